# Initial kernel scaffold; baseline (speedup 1.0000x reference)
import jax
import jax.numpy as jnp
from jax.experimental import pallas as pl


def kernel(seq_hiddens, pool_hiddens, span_mask, combine_W, combine_b, lstm_Wih, lstm_Whh, lstm_bih, lstm_bhh, Wq, bq, Wk, bk, Wv, bv, Wo, bo, ln_g, ln_b, cls_W, cls_b, rel_emb, su_W, su_b, sv_W, sv_b, suv_W, suv_b):
    raise NotImplementedError("write your pallas kernel here")


# speedup vs baseline: 3.3107x; 3.3107x over previous
import numpy as np
import jax
import jax.numpy as jnp
from jax.experimental import pallas as pl
from jax.experimental.pallas import tpu as pltpu

B, S, H, W = 2, 64, 768, 6
NH, DH = 12, 64
R, TAGS = 12, 2
L = S * W - W * (W - 1) // 2
LP = 384
G4 = 4 * H
NEG = -1e30
LN_EPS = 1e-12
TI = 8


def _span_gather_idx():
    idx = []
    for i in range(S):
        for w in range(1, min(W, S - i) + 1):
            idx.append((w - 1) * S + i)
    while len(idx) < LP:
        idx.append(0)
    return np.asarray(idx, np.int32)


_GATHER = _span_gather_idx()


def _spans_body(seq_ref, m_ref):
    seq = seq_ref[0]
    cur = seq
    m_ref[0, 0] = cur
    for w in range(1, W):
        shifted = jnp.concatenate(
            [seq[w:, :], jnp.full((w, H), NEG, seq.dtype)], axis=0)
        cur = jnp.maximum(cur, shifted)
        m_ref[0, w] = cur


def _gates_body(sp_ref, pool_ref, wcs_ref, wcp_ref, cb_ref, wih_ref, bg_ref,
                g_ref):
    sp = sp_ref[0]
    x = jnp.tanh(
        jnp.dot(sp, wcs_ref[...], preferred_element_type=jnp.float32)
        + jnp.dot(pool_ref[0], wcp_ref[...], preferred_element_type=jnp.float32)
        + cb_ref[...])
    g_ref[0, :, 0, :] = (
        jnp.dot(x, wih_ref[...], preferred_element_type=jnp.float32)
        + bg_ref[...])


def _lstm_body(g_ref, whh_ref, ys_ref):
    def step(l, carry):
        h, c = carry
        grow = g_ref[0, pl.ds(l, 1), 0, :]
        g = grow + jnp.dot(h, whh_ref[...], preferred_element_type=jnp.float32)
        i = jax.nn.sigmoid(g[:, :H])
        f = jax.nn.sigmoid(g[:, H:2 * H])
        gg = jnp.tanh(g[:, 2 * H:3 * H])
        o = jax.nn.sigmoid(g[:, 3 * H:])
        c = f * c + i * gg
        h = o * jnp.tanh(c)
        ys_ref[0, pl.ds(l, 1), 0, :] = h
        return (h, c)

    h0 = jnp.zeros((1, H), jnp.float32)
    jax.lax.fori_loop(0, LP, step, (h0, h0))


def _attn_body(x_ref, amask_ref, wq_ref, wk_ref, wv_ref, wo_ref, bq_ref,
               bk_ref, bv_ref, bo_ref, lng_ref, lnb_ref, attn_ref,
               q_scr, k_scr, v_scr, ctx_scr):
    x = x_ref[0]
    q_scr[...] = jnp.dot(x, wq_ref[...], preferred_element_type=jnp.float32) + bq_ref[...]
    k_scr[...] = jnp.dot(x, wk_ref[...], preferred_element_type=jnp.float32) + bk_ref[...]
    v_scr[...] = jnp.dot(x, wv_ref[...], preferred_element_type=jnp.float32) + bv_ref[...]
    amask = amask_ref[0]
    for h in range(NH):
        sl = slice(h * DH, (h + 1) * DH)
        qh = q_scr[:, sl]
        kh = k_scr[:, sl]
        vh = v_scr[:, sl]
        s = jax.lax.dot_general(
            qh, kh, (((1,), (1,)), ((), ())),
            preferred_element_type=jnp.float32) * 0.125 + amask
        m = jnp.max(s, axis=-1, keepdims=True)
        p = jnp.exp(s - m)
        p = p / jnp.sum(p, axis=-1, keepdims=True)
        ctx_scr[:, sl] = jnp.dot(p, vh, preferred_element_type=jnp.float32)
    o = (jnp.dot(ctx_scr[...], wo_ref[...], preferred_element_type=jnp.float32)
         + bo_ref[...] + x)
    mu = jnp.mean(o, axis=-1, keepdims=True)
    d = o - mu
    var = jnp.mean(d * d, axis=-1, keepdims=True)
    attn_ref[0] = d * jax.lax.rsqrt(var + LN_EPS) * lng_ref[...] + lnb_ref[...]


def _heads_body(attn_ref, clsw_ref, clsb_ref, sut_ref, sub_ref, svt_ref,
                svb_ref, suv1_ref, suv2_ref, suvb_ref, cls_ref, a_ref, b_ref):
    attn = attn_ref[0]
    cls_ref[0] = (jnp.dot(attn, clsw_ref[...], preferred_element_type=jnp.float32)
                  + clsb_ref[...])
    u = jnp.tanh(jnp.dot(attn, sut_ref[...], preferred_element_type=jnp.float32)
                 + sub_ref[...])
    vv = jnp.tanh(jnp.dot(attn, svt_ref[...], preferred_element_type=jnp.float32)
                  + svb_ref[...])
    a_ref[0] = (jnp.dot(u, suv1_ref[...], preferred_element_type=jnp.float32)
                + suvb_ref[...])
    b_ref[0] = jnp.dot(vv, suv2_ref[...], preferred_element_type=jnp.float32)


def _sel_body(a_ref, b_ref, rel_ref, out_ref):
    rows = b_ref[0]
    for ii in range(TI):
        t = jnp.tanh(a_ref[0] + rows[ii:ii + 1, :])
        st = jax.lax.dot_general(
            rel_ref[...], t, (((1,), (1,)), ((), ())),
            preferred_element_type=jnp.float32)
        out_ref[0, ii] = st[:, :L]


def kernel(seq_hiddens, pool_hiddens, span_mask, combine_W, combine_b,
           lstm_Wih, lstm_Whh, lstm_bih, lstm_bhh, Wq, bq, Wk, bk, Wv, bv,
           Wo, bo, ln_g, ln_b, cls_W, cls_b, rel_emb, su_W, su_b, sv_W,
           sv_b, suv_W, suv_b):
    f32 = jnp.float32
    par1 = pltpu.CompilerParams(dimension_semantics=("parallel",),
                                vmem_limit_bytes=100 * 1024 * 1024)
    par2 = pltpu.CompilerParams(dimension_semantics=("parallel", "arbitrary"),
                                vmem_limit_bytes=100 * 1024 * 1024)

    m = pl.pallas_call(
        _spans_body,
        grid=(B,),
        in_specs=[pl.BlockSpec((1, S, H), lambda b: (b, 0, 0))],
        out_specs=pl.BlockSpec((1, W, S, H), lambda b: (b, 0, 0, 0)),
        out_shape=jax.ShapeDtypeStruct((B, W, S, H), f32),
        compiler_params=par1,
    )(seq_hiddens)

    spans = jnp.take(m.reshape(B, W * S, H), _GATHER, axis=1)

    wcsT = combine_W[:, :H].T
    wcpT = combine_W[:, H:].T
    bias_g = (lstm_bih + lstm_bhh).reshape(1, G4)
    gates = pl.pallas_call(
        _gates_body,
        grid=(B,),
        in_specs=[
            pl.BlockSpec((1, LP, H), lambda b: (b, 0, 0)),
            pl.BlockSpec((1, 1, H), lambda b: (b, 0, 0)),
            pl.BlockSpec((H, H), lambda b: (0, 0)),
            pl.BlockSpec((H, H), lambda b: (0, 0)),
            pl.BlockSpec((1, H), lambda b: (0, 0)),
            pl.BlockSpec((H, G4), lambda b: (0, 0)),
            pl.BlockSpec((1, G4), lambda b: (0, 0)),
        ],
        out_specs=pl.BlockSpec((1, LP, 1, G4), lambda b: (b, 0, 0, 0)),
        out_shape=jax.ShapeDtypeStruct((B, LP, 1, G4), f32),
        compiler_params=par1,
    )(spans, pool_hiddens.reshape(B, 1, H), wcsT, wcpT,
      combine_b.reshape(1, H), lstm_Wih.T, bias_g)

    ys4 = pl.pallas_call(
        _lstm_body,
        grid=(B,),
        in_specs=[
            pl.BlockSpec((1, LP, 1, G4), lambda b: (b, 0, 0, 0)),
            pl.BlockSpec((H, G4), lambda b: (0, 0)),
        ],
        out_specs=pl.BlockSpec((1, LP, 1, H), lambda b: (b, 0, 0, 0)),
        out_shape=jax.ShapeDtypeStruct((B, LP, 1, H), f32),
        compiler_params=par1,
    )(gates, lstm_Whh.T)
    x = ys4.reshape(B, LP, H)

    col = np.arange(LP) < L
    sm_pad = jnp.pad(span_mask, ((0, 0), (0, LP - L)))
    amask = jnp.where(col[None, :], (1.0 - sm_pad) * -10000.0, -1e9)
    attn = pl.pallas_call(
        _attn_body,
        grid=(B,),
        in_specs=[
            pl.BlockSpec((1, LP, H), lambda b: (b, 0, 0)),
            pl.BlockSpec((1, 1, LP), lambda b: (b, 0, 0)),
            pl.BlockSpec((H, H), lambda b: (0, 0)),
            pl.BlockSpec((H, H), lambda b: (0, 0)),
            pl.BlockSpec((H, H), lambda b: (0, 0)),
            pl.BlockSpec((H, H), lambda b: (0, 0)),
            pl.BlockSpec((1, H), lambda b: (0, 0)),
            pl.BlockSpec((1, H), lambda b: (0, 0)),
            pl.BlockSpec((1, H), lambda b: (0, 0)),
            pl.BlockSpec((1, H), lambda b: (0, 0)),
            pl.BlockSpec((1, H), lambda b: (0, 0)),
            pl.BlockSpec((1, H), lambda b: (0, 0)),
        ],
        out_specs=pl.BlockSpec((1, LP, H), lambda b: (b, 0, 0)),
        out_shape=jax.ShapeDtypeStruct((B, LP, H), f32),
        scratch_shapes=[pltpu.VMEM((LP, H), f32)] * 4,
        compiler_params=par1,
    )(x, amask.reshape(B, 1, LP), Wq.T, Wk.T, Wv.T, Wo.T,
      bq.reshape(1, H), bk.reshape(1, H), bv.reshape(1, H), bo.reshape(1, H),
      ln_g.reshape(1, H), ln_b.reshape(1, H))

    CLSP = 128
    cls_wp = jnp.zeros((CLSP, H), f32).at[:TAGS].set(cls_W)
    cls_bp = jnp.zeros((1, CLSP), f32).at[0, :TAGS].set(cls_b)
    cls_pad, a_half, b_half = pl.pallas_call(
        _heads_body,
        grid=(B,),
        in_specs=[
            pl.BlockSpec((1, LP, H), lambda b: (b, 0, 0)),
            pl.BlockSpec((H, CLSP), lambda b: (0, 0)),
            pl.BlockSpec((1, CLSP), lambda b: (0, 0)),
            pl.BlockSpec((H, H), lambda b: (0, 0)),
            pl.BlockSpec((1, H), lambda b: (0, 0)),
            pl.BlockSpec((H, H), lambda b: (0, 0)),
            pl.BlockSpec((1, H), lambda b: (0, 0)),
            pl.BlockSpec((H, H), lambda b: (0, 0)),
            pl.BlockSpec((H, H), lambda b: (0, 0)),
            pl.BlockSpec((1, H), lambda b: (0, 0)),
        ],
        out_specs=(
            pl.BlockSpec((1, LP, CLSP), lambda b: (b, 0, 0)),
            pl.BlockSpec((1, LP, H), lambda b: (b, 0, 0)),
            pl.BlockSpec((1, LP, H), lambda b: (b, 0, 0)),
        ),
        out_shape=(
            jax.ShapeDtypeStruct((B, LP, CLSP), f32),
            jax.ShapeDtypeStruct((B, LP, H), f32),
            jax.ShapeDtypeStruct((B, LP, H), f32),
        ),
        compiler_params=par1,
    )(attn, cls_wp.T, cls_bp, su_W.T, su_b.reshape(1, H), sv_W.T,
      sv_b.reshape(1, H), suv_W[:, :H].T, suv_W[:, H:].T, suv_b.reshape(1, H))

    sel_pad = pl.pallas_call(
        _sel_body,
        grid=(B, LP // TI),
        in_specs=[
            pl.BlockSpec((1, LP, H), lambda b, i: (b, 0, 0)),
            pl.BlockSpec((1, TI, H), lambda b, i: (b, i, 0)),
            pl.BlockSpec((R, H), lambda b, i: (0, 0)),
        ],
        out_specs=pl.BlockSpec((1, TI, R, L), lambda b, i: (b, i, 0, 0)),
        out_shape=jax.ShapeDtypeStruct((B, LP, R, L), f32),
        compiler_params=par2,
    )(a_half, b_half, rel_emb)

    sel = sel_pad[:, :L]
    classify = cls_pad[:, :L, :TAGS]
    return sel, classify


# speedup vs baseline: 3.4862x; 1.0530x over previous
import numpy as np
import jax
import jax.numpy as jnp
from jax.experimental import pallas as pl
from jax.experimental.pallas import tpu as pltpu

B, S, H, W = 2, 64, 768, 6
NH, DH = 12, 64
R, TAGS = 12, 2
L = S * W - W * (W - 1) // 2
LP = 384
G4 = 4 * H
NEG = -1e30
LN_EPS = 1e-12
TI = 8


def _span_perm():
    p = np.zeros((LP, W * S), np.float32)
    l = 0
    for i in range(S):
        for w in range(1, min(W, S - i) + 1):
            p[l, (w - 1) * S + i] = 1.0
            l += 1
    return p


_PERM = _span_perm()


def _dot_t(x, w):
    return jax.lax.dot_general(
        x, w, (((1,), (1,)), ((), ())), preferred_element_type=jnp.float32)


def _front_body(seq_ref, perm_ref, pool_ref, wc_ref, cb_ref, wih_ref, bg_ref,
                g_ref):
    seq = seq_ref[0]
    cur = seq
    ms = [cur]
    for w in range(1, W):
        shifted = jnp.concatenate(
            [seq[w:, :], jnp.full((w, H), NEG, seq.dtype)], axis=0)
        cur = jnp.maximum(cur, shifted)
        ms.append(cur)
    m = jnp.concatenate(ms, axis=0)
    sp = jnp.dot(perm_ref[...], m, preferred_element_type=jnp.float32)
    x = jnp.tanh(_dot_t(sp, wc_ref[:, :H])
                 + _dot_t(pool_ref[0], wc_ref[:, H:])
                 + cb_ref[...])
    g_ref[0, :, 0, :] = _dot_t(x, wih_ref[...]) + bg_ref[...]


def _lstm_body(g_ref, whh_ref, ys_ref):
    def step(l, carry):
        h, c = carry
        grow = g_ref[0, pl.ds(l, 1), 0, :]
        g = grow + jnp.dot(h, whh_ref[...], preferred_element_type=jnp.float32)
        i = jax.nn.sigmoid(g[:, :H])
        f = jax.nn.sigmoid(g[:, H:2 * H])
        gg = jnp.tanh(g[:, 2 * H:3 * H])
        o = jax.nn.sigmoid(g[:, 3 * H:])
        c = f * c + i * gg
        h = o * jnp.tanh(c)
        ys_ref[0, pl.ds(l, 1), 0, :] = h
        return (h, c)

    h0 = jnp.zeros((1, H), jnp.float32)
    jax.lax.fori_loop(0, LP, step, (h0, h0))


def _attn_body(x_ref, amask_ref, wq_ref, wk_ref, wv_ref, wo_ref, bq_ref,
               bk_ref, bv_ref, bo_ref, lng_ref, lnb_ref, attn_ref,
               q_scr, k_scr, v_scr, ctx_scr):
    x = x_ref[0]
    q_scr[...] = _dot_t(x, wq_ref[...]) + bq_ref[...]
    k_scr[...] = _dot_t(x, wk_ref[...]) + bk_ref[...]
    v_scr[...] = _dot_t(x, wv_ref[...]) + bv_ref[...]
    amask = amask_ref[0]
    for h in range(NH):
        sl = slice(h * DH, (h + 1) * DH)
        qh = q_scr[:, sl]
        kh = k_scr[:, sl]
        vh = v_scr[:, sl]
        s = jax.lax.dot_general(
            qh, kh, (((1,), (1,)), ((), ())),
            preferred_element_type=jnp.float32) * 0.125 + amask
        m = jnp.max(s, axis=-1, keepdims=True)
        p = jnp.exp(s - m)
        p = p / jnp.sum(p, axis=-1, keepdims=True)
        ctx_scr[:, sl] = jnp.dot(p, vh, preferred_element_type=jnp.float32)
    o = _dot_t(ctx_scr[...], wo_ref[...]) + bo_ref[...] + x
    mu = jnp.mean(o, axis=-1, keepdims=True)
    d = o - mu
    var = jnp.mean(d * d, axis=-1, keepdims=True)
    attn_ref[0] = d * jax.lax.rsqrt(var + LN_EPS) * lng_ref[...] + lnb_ref[...]


def _heads_body(attn_ref, clsw_ref, clsb_ref, suw_ref, sub_ref, svw_ref,
                svb_ref, suvw_ref, suvb_ref, cls_ref, a_ref, b_ref):
    attn = attn_ref[0]
    cls_ref[0] = _dot_t(attn, clsw_ref[...]) + clsb_ref[...]
    u = jnp.tanh(_dot_t(attn, suw_ref[...]) + sub_ref[...])
    vv = jnp.tanh(_dot_t(attn, svw_ref[...]) + svb_ref[...])
    a_ref[0] = _dot_t(u, suvw_ref[:, :H]) + suvb_ref[...]
    b_ref[0] = _dot_t(vv, suvw_ref[:, H:])


def _sel_body(a_ref, b_ref, rel_ref, out_ref):
    rows = b_ref[0]
    for ii in range(TI):
        t = jnp.tanh(a_ref[0] + rows[ii:ii + 1, :])
        st = jax.lax.dot_general(
            rel_ref[...], t, (((1,), (1,)), ((), ())),
            preferred_element_type=jnp.float32)
        out_ref[0, ii] = st[:, :L]


def kernel(seq_hiddens, pool_hiddens, span_mask, combine_W, combine_b,
           lstm_Wih, lstm_Whh, lstm_bih, lstm_bhh, Wq, bq, Wk, bk, Wv, bv,
           Wo, bo, ln_g, ln_b, cls_W, cls_b, rel_emb, su_W, su_b, sv_W,
           sv_b, suv_W, suv_b):
    f32 = jnp.float32
    par1 = pltpu.CompilerParams(dimension_semantics=("parallel",),
                                vmem_limit_bytes=100 * 1024 * 1024)
    par2 = pltpu.CompilerParams(dimension_semantics=("parallel", "arbitrary"),
                                vmem_limit_bytes=100 * 1024 * 1024)

    bias_g = (lstm_bih + lstm_bhh).reshape(1, G4)
    gates = pl.pallas_call(
        _front_body,
        grid=(B,),
        in_specs=[
            pl.BlockSpec((1, S, H), lambda b: (b, 0, 0)),
            pl.BlockSpec((LP, W * S), lambda b: (0, 0)),
            pl.BlockSpec((1, 1, H), lambda b: (b, 0, 0)),
            pl.BlockSpec((H, 2 * H), lambda b: (0, 0)),
            pl.BlockSpec((1, H), lambda b: (0, 0)),
            pl.BlockSpec((G4, H), lambda b: (0, 0)),
            pl.BlockSpec((1, G4), lambda b: (0, 0)),
        ],
        out_specs=pl.BlockSpec((1, LP, 1, G4), lambda b: (b, 0, 0, 0)),
        out_shape=jax.ShapeDtypeStruct((B, LP, 1, G4), f32),
        compiler_params=par1,
    )(seq_hiddens, jnp.asarray(_PERM), pool_hiddens.reshape(B, 1, H),
      combine_W, combine_b.reshape(1, H), lstm_Wih, bias_g)

    ys4 = pl.pallas_call(
        _lstm_body,
        grid=(B,),
        in_specs=[
            pl.BlockSpec((1, LP, 1, G4), lambda b: (b, 0, 0, 0)),
            pl.BlockSpec((H, G4), lambda b: (0, 0)),
        ],
        out_specs=pl.BlockSpec((1, LP, 1, H), lambda b: (b, 0, 0, 0)),
        out_shape=jax.ShapeDtypeStruct((B, LP, 1, H), f32),
        compiler_params=par1,
    )(gates, lstm_Whh.T)
    x = ys4.reshape(B, LP, H)

    col = np.arange(LP) < L
    sm_pad = jnp.pad(span_mask, ((0, 0), (0, LP - L)))
    amask = jnp.where(col[None, :], (1.0 - sm_pad) * -10000.0, -1e9)
    attn = pl.pallas_call(
        _attn_body,
        grid=(B,),
        in_specs=[
            pl.BlockSpec((1, LP, H), lambda b: (b, 0, 0)),
            pl.BlockSpec((1, 1, LP), lambda b: (b, 0, 0)),
            pl.BlockSpec((H, H), lambda b: (0, 0)),
            pl.BlockSpec((H, H), lambda b: (0, 0)),
            pl.BlockSpec((H, H), lambda b: (0, 0)),
            pl.BlockSpec((H, H), lambda b: (0, 0)),
            pl.BlockSpec((1, H), lambda b: (0, 0)),
            pl.BlockSpec((1, H), lambda b: (0, 0)),
            pl.BlockSpec((1, H), lambda b: (0, 0)),
            pl.BlockSpec((1, H), lambda b: (0, 0)),
            pl.BlockSpec((1, H), lambda b: (0, 0)),
            pl.BlockSpec((1, H), lambda b: (0, 0)),
        ],
        out_specs=pl.BlockSpec((1, LP, H), lambda b: (b, 0, 0)),
        out_shape=jax.ShapeDtypeStruct((B, LP, H), f32),
        scratch_shapes=[pltpu.VMEM((LP, H), f32)] * 4,
        compiler_params=par1,
    )(x, amask.reshape(B, 1, LP), Wq, Wk, Wv, Wo,
      bq.reshape(1, H), bk.reshape(1, H), bv.reshape(1, H), bo.reshape(1, H),
      ln_g.reshape(1, H), ln_b.reshape(1, H))

    CLSP = 128
    cls_wp = jnp.zeros((CLSP, H), f32).at[:TAGS].set(cls_W)
    cls_bp = jnp.zeros((1, CLSP), f32).at[0, :TAGS].set(cls_b)
    cls_pad, a_half, b_half = pl.pallas_call(
        _heads_body,
        grid=(B,),
        in_specs=[
            pl.BlockSpec((1, LP, H), lambda b: (b, 0, 0)),
            pl.BlockSpec((CLSP, H), lambda b: (0, 0)),
            pl.BlockSpec((1, CLSP), lambda b: (0, 0)),
            pl.BlockSpec((H, H), lambda b: (0, 0)),
            pl.BlockSpec((1, H), lambda b: (0, 0)),
            pl.BlockSpec((H, H), lambda b: (0, 0)),
            pl.BlockSpec((1, H), lambda b: (0, 0)),
            pl.BlockSpec((H, 2 * H), lambda b: (0, 0)),
            pl.BlockSpec((1, H), lambda b: (0, 0)),
        ],
        out_specs=(
            pl.BlockSpec((1, LP, CLSP), lambda b: (b, 0, 0)),
            pl.BlockSpec((1, LP, H), lambda b: (b, 0, 0)),
            pl.BlockSpec((1, LP, H), lambda b: (b, 0, 0)),
        ),
        out_shape=(
            jax.ShapeDtypeStruct((B, LP, CLSP), f32),
            jax.ShapeDtypeStruct((B, LP, H), f32),
            jax.ShapeDtypeStruct((B, LP, H), f32),
        ),
        compiler_params=par1,
    )(attn, cls_wp, cls_bp, su_W, su_b.reshape(1, H), sv_W,
      sv_b.reshape(1, H), suv_W, suv_b.reshape(1, H))

    sel_pad = pl.pallas_call(
        _sel_body,
        grid=(B, LP // TI),
        in_specs=[
            pl.BlockSpec((1, LP, H), lambda b, i: (b, 0, 0)),
            pl.BlockSpec((1, TI, H), lambda b, i: (b, i, 0)),
            pl.BlockSpec((R, H), lambda b, i: (0, 0)),
        ],
        out_specs=pl.BlockSpec((1, TI, R, L), lambda b, i: (b, i, 0, 0)),
        out_shape=jax.ShapeDtypeStruct((B, LP, R, L), f32),
        compiler_params=par2,
    )(a_half, b_half, rel_emb)

    sel = sel_pad[:, :L]
    classify = cls_pad[:, :L, :TAGS]
    return sel, classify
